# baseline (speedup 1.0000x reference)
import functools

import numpy as np
import jax
import jax.numpy as jnp
from jax.experimental import pallas as pl
from jax.experimental.pallas import tpu as pltpu

_TV = 7168
_B = 128
_N = 10

_TINY = np.float32(np.finfo(np.float32).tiny)


def _threefry2x32(k0, k1, x0, x1, off=np.uint32(0)):
    def rotl(x, r):
        return (x << jnp.uint32(r)) | (x >> jnp.uint32(32 - r))

    ks2 = k0 ^ k1 ^ jnp.uint32(0x1BD11BDA)
    ks = (k0, k1, ks2)
    rots = ((13, 15, 26, 6), (17, 29, 16, 24))
    x0 = x0 + ks[0]
    x1 = x1 + (ks[1] + off)
    for i in range(5):
        for r in rots[i % 2]:
            x0 = x0 + x1
            x1 = rotl(x1, r)
            x1 = x1 ^ x0
        x0 = x0 + ks[(i + 1) % 3]
        x1 = x1 + (ks[(i + 2) % 3] + jnp.uint32(i + 1))
    return x0, x1


def _gumbel_at(k0, k1, cnt_u32, off=np.uint32(0)):
    o0, o1 = _threefry2x32(k0, k1, jnp.uint32(0), cnt_u32, off)
    bits = o0 ^ o1
    fb = (bits >> jnp.uint32(9)) | jnp.uint32(0x3F800000)
    f = jax.lax.bitcast_convert_type(fb, jnp.float32) - jnp.float32(1.0)
    u = jnp.maximum(_TINY, f + _TINY)
    return -jnp.log(-jnp.log(u))


def _pick_at(onehot, arr, neg_inf):
    return jnp.max(jnp.where(onehot, arr, neg_inf), axis=1, keepdims=True)


def _body(xinit_ref, ub_ref, thrb_ref, keys_ref,
          emb_ref, w_ref, b_ref, tv_ref,
          out_ref,
          mx_ref, se_ref, s1_ref, l1_ref, i1_ref, s2_ref, l2_ref, i2_ref,
          h_ref, xs_ref, x1v_ref, x1s_ref, cnt0_ref, g_ref, sem, *, V, T):
    i = pl.program_id(0)
    j = pl.program_id(1)
    neg_inf = jnp.float32(-jnp.inf)

    @pl.when(j == 0)
    def _prologue():
        tv = tv_ref[0]

        @pl.when(i == 0)
        def _first():
            row = jax.lax.broadcasted_iota(jnp.int32, (_B, _TV), 0)
            lane = jax.lax.broadcasted_iota(jnp.int32, (_B, _TV), 1)
            cnt0_ref[...] = row * V + lane

            def loop0(b, _):
                xs_ref[b, 0] = xinit_ref[b]
                return _
            jax.lax.fori_loop(0, _B, loop0, None)

        @pl.when(i > 0)
        def _update():
            def loop(b, _):
                xp = xs_ref[b, 0]
                x1 = x1s_ref[b, 0]
                jump = jnp.logical_and(ub_ref[i - 1, b] < thrb_ref[i - 1],
                                       x1 != xp)
                xs_ref[b, 0] = jnp.where(jump, x1, xp)
                return _
            jax.lax.fori_loop(0, _B, loop, None)

        def gather(b, _):
            pltpu.make_async_copy(
                emb_ref.at[pl.ds(xs_ref[b, 0], 1), :],
                h_ref.at[pl.ds(b, 1), :], sem).start()
            return _
        jax.lax.fori_loop(0, _B, gather, None)

        def drain(b, _):
            pltpu.make_async_copy(
                emb_ref.at[pl.ds(0, 1), :],
                h_ref.at[pl.ds(b, 1), :], sem).wait()
            return _
        jax.lax.fori_loop(0, _B, drain, None)
        h_ref[...] = h_ref[...] + tv

        mx_ref[...] = jnp.full((_B, 1), neg_inf, jnp.float32)
        se_ref[...] = jnp.zeros((_B, 1), jnp.float32)
        s1_ref[...] = jnp.full((_B, 1), neg_inf, jnp.float32)
        s2_ref[...] = jnp.full((_B, 1), neg_inf, jnp.float32)
        l1_ref[...] = jnp.zeros((_B, 1), jnp.float32)
        l2_ref[...] = jnp.zeros((_B, 1), jnp.float32)
        i1_ref[...] = jnp.zeros((_B, 1), jnp.int32)
        i2_ref[...] = jnp.zeros((_B, 1), jnp.int32)

    @pl.when(j < T)
    def _sweep():
        cnt0 = cnt0_ref[...]
        g_ref[...] = _gumbel_at(keys_ref[i, 0], keys_ref[i, 1],
                                cnt0.astype(jnp.uint32),
                                j.astype(jnp.uint32) * np.uint32(_TV))

        logits = jnp.dot(h_ref[...], w_ref[...],
                         preferred_element_type=jnp.float32) + b_ref[...]

        tmax = jnp.max(logits, axis=1, keepdims=True)
        old_mx = mx_ref[...]
        new_mx = jnp.maximum(old_mx, tmax)
        e = jnp.exp(logits - new_mx)
        se_ref[...] = se_ref[...] * jnp.exp(old_mx - new_mx) \
            + jnp.sum(e, axis=1, keepdims=True)
        mx_ref[...] = new_mx

        s = g_ref[...] + logits
        rowv = jax.lax.broadcasted_iota(jnp.int32, (_B, 1), 0) * V
        big = jnp.int32(0x7FFFFFFF)
        m1 = jnp.max(s, axis=1, keepdims=True)
        a1abs = jnp.min(jnp.where(s == m1, cnt0, big), axis=1, keepdims=True)
        oh1 = cnt0 == a1abs
        lt1 = _pick_at(oh1, logits, neg_inf)
        s_m = jnp.where(oh1, neg_inf, s)
        m2 = jnp.max(s_m, axis=1, keepdims=True)
        a2abs = jnp.min(jnp.where(s_m == m2, cnt0, big), axis=1, keepdims=True)
        lt2 = _pick_at(cnt0 == a2abs, logits, neg_inf)
        ga1 = a1abs - rowv + j * _TV
        ga2 = a2abs - rowv + j * _TV

        rs1, rs2 = s1_ref[...], s2_ref[...]
        take1 = m1 > rs1
        n1_s = jnp.where(take1, m1, rs1)
        n1_l = jnp.where(take1, lt1, l1_ref[...])
        n1_i = jnp.where(take1, ga1, i1_ref[...])
        c_s = jnp.where(take1, rs1, rs2)
        c_l = jnp.where(take1, l1_ref[...], l2_ref[...])
        c_i = jnp.where(take1, i1_ref[...], i2_ref[...])
        d_s = jnp.where(take1, m2, m1)
        d_l = jnp.where(take1, lt2, lt1)
        d_i = jnp.where(take1, ga2, ga1)
        take2 = d_s > c_s
        s2_ref[...] = jnp.where(take2, d_s, c_s)
        l2_ref[...] = jnp.where(take2, d_l, c_l)
        i2_ref[...] = jnp.where(take2, d_i, c_i)
        s1_ref[...] = n1_s
        l1_ref[...] = n1_l
        i1_ref[...] = n1_i

    @pl.when(j == T)
    def _finalize():
        mx = mx_ref[...]
        se = se_ref[...]
        eps = jnp.float32(1e-30)
        rowv = jax.lax.broadcasted_iota(jnp.int32, (_B, 1), 0)
        i1 = i1_ref[...]
        i2 = i2_ref[...]
        k0 = keys_ref[i, 0]
        k1 = keys_ref[i, 1]
        g1 = _gumbel_at(k0, k1, (rowv * V + i1).astype(jnp.uint32))
        g2 = _gumbel_at(k0, k1, (rowv * V + i2).astype(jnp.uint32))
        v1 = jnp.log(jnp.exp(l1_ref[...] - mx) / se + eps) + g1
        v2 = jnp.log(jnp.exp(l2_ref[...] - mx) / se + eps) + g2
        pick2 = (v2 > v1) | ((v2 == v1) & (i2 < i1))
        x1 = jnp.where(pick2, i2, i1)

        @pl.when(i < _N - 1)
        def _handoff():
            x1v_ref[...] = x1
            dma = pltpu.make_async_copy(x1v_ref, x1s_ref, sem)
            dma.start()
            dma.wait()

        @pl.when(i == _N - 1)
        def _emit():
            out_ref[...] = x1


def kernel(x_init, emb, W, b, t_w, n_steps):
    B, S = x_init.shape
    V, D = emb.shape
    step_size = 1.0 / _N
    t_disc = jnp.array([step_size * i for i in range(_N)] + [1.0],
                       dtype=jnp.float32)
    t_disc = t_disc * (n_steps / _N)

    T = -(-V // _TV)
    VP = T * _TV
    w_pad = jnp.pad(W, ((0, 0), (0, VP - V)))
    b_pad = jnp.pad(b, (0, VP - V),
                    constant_values=-jnp.inf).reshape(1, VP)

    key = jax.random.key(42)
    keys = []
    unifs = []
    thrs = []
    for i in range(_N):
        key, k_cat, k_jump, k_cat2 = jax.random.split(key, 4)
        keys.append(jax.random.key_data(k_cat))
        if i < _N - 1:
            t = t_disc[i]
            h = t_disc[i + 1] - t_disc[i]
            unifs.append(jax.random.uniform(k_jump, (B, S)).reshape(B))
            intensity = jnp.float32(1.0) / (1.0 - t)
            thrs.append(1.0 - jnp.exp(-h * intensity))
    keys = jnp.stack(keys)
    ub = jax.lax.bitcast_convert_type(
        jnp.stack(unifs + [jnp.zeros(B, jnp.float32)]), jnp.int32)
    thrb = jax.lax.bitcast_convert_type(
        jnp.stack(thrs + [jnp.float32(0.0)]), jnp.int32)
    tvec = (t_disc[:_N, None] * t_w[None, :]).reshape(_N, 1, D)

    f32 = jnp.float32
    out = pl.pallas_call(
        functools.partial(_body, V=V, T=T),
        grid=(_N, T + 1),
        in_specs=[
            pl.BlockSpec(memory_space=pltpu.SMEM),
            pl.BlockSpec(memory_space=pltpu.SMEM),
            pl.BlockSpec(memory_space=pltpu.SMEM),
            pl.BlockSpec(memory_space=pltpu.SMEM),
            pl.BlockSpec(memory_space=pltpu.MemorySpace.HBM),
            pl.BlockSpec((D, _TV),
                         lambda i, j: (0, jnp.minimum(j, T - 1))),
            pl.BlockSpec((1, _TV),
                         lambda i, j: (0, jnp.minimum(j, T - 1))),
            pl.BlockSpec((1, 1, D), lambda i, j: (i, 0, 0)),
        ],
        out_specs=pl.BlockSpec((_B, 1), lambda i, j: (0, 0)),
        out_shape=jax.ShapeDtypeStruct((_B, 1), jnp.int32),
        scratch_shapes=[
            pltpu.VMEM((_B, 1), f32), pltpu.VMEM((_B, 1), f32),
            pltpu.VMEM((_B, 1), f32), pltpu.VMEM((_B, 1), f32),
            pltpu.VMEM((_B, 1), jnp.int32),
            pltpu.VMEM((_B, 1), f32), pltpu.VMEM((_B, 1), f32),
            pltpu.VMEM((_B, 1), jnp.int32),
            pltpu.VMEM((_B, D), f32),
            pltpu.SMEM((_B, 1), jnp.int32),
            pltpu.VMEM((_B, 1), jnp.int32),
            pltpu.SMEM((_B, 1), jnp.int32),
            pltpu.VMEM((_B, _TV), jnp.int32),
            pltpu.VMEM((_B, _TV), jnp.float32),
            pltpu.SemaphoreType.DMA,
        ],
        compiler_params=pltpu.CompilerParams(
            dimension_semantics=("arbitrary", "arbitrary"),
            vmem_limit_bytes=60 * 1024 * 1024),
    )(x_init[:, 0], ub, thrb, keys, emb, w_pad, b_pad, tvec)
    return out


# speedup vs baseline: 1.0039x; 1.0039x over previous
import functools

import numpy as np
import jax
import jax.numpy as jnp
from jax.experimental import pallas as pl
from jax.experimental.pallas import tpu as pltpu

_TV = 7168
_B = 128
_N = 10

_TINY = np.float32(np.finfo(np.float32).tiny)


def _threefry2x32(k0, k1, x0, x1, off=np.uint32(0)):
    def rotl(x, r):
        return (x << jnp.uint32(r)) | (x >> jnp.uint32(32 - r))

    ks2 = k0 ^ k1 ^ jnp.uint32(0x1BD11BDA)
    ks = (k0, k1, ks2)
    rots = ((13, 15, 26, 6), (17, 29, 16, 24))
    x0 = x0 + ks[0]
    x1 = x1 + (ks[1] + off)
    for i in range(5):
        for r in rots[i % 2]:
            x0 = x0 + x1
            x1 = rotl(x1, r)
            x1 = x1 ^ x0
        x0 = x0 + ks[(i + 1) % 3]
        x1 = x1 + (ks[(i + 2) % 3] + jnp.uint32(i + 1))
    return x0, x1


def _gumbel_at(k0, k1, cnt_u32, off=np.uint32(0)):
    o0, o1 = _threefry2x32(k0, k1, jnp.uint32(0), cnt_u32, off)
    bits = o0 ^ o1
    fb = (bits >> jnp.uint32(9)) | jnp.uint32(0x3F800000)
    f = jax.lax.bitcast_convert_type(fb, jnp.float32) - jnp.float32(1.0)
    u = jnp.maximum(_TINY, f + _TINY)
    return -jnp.log(-jnp.log(u))


def _pick_at(onehot, arr, neg_inf):
    return jnp.max(jnp.where(onehot, arr, neg_inf), axis=1, keepdims=True)


def _body(xinit_ref, ub_ref, thrb_ref, keys_ref,
          emb_ref, w_ref, b_ref, tv_ref,
          out_ref,
          mx_ref, se_ref, s1_ref, l1_ref, i1_ref, s2_ref, l2_ref, i2_ref,
          h_ref, xs_ref, x1v_ref, x1s_ref, cnt0_ref, g_ref, sem, *, V, T):
    i = pl.program_id(0)
    j = pl.program_id(1)
    neg_inf = jnp.float32(-jnp.inf)

    @pl.when(j == 0)
    def _prologue():
        tv = tv_ref[0]

        @pl.when(i == 0)
        def _first():
            row = jax.lax.broadcasted_iota(jnp.int32, (_B, _TV), 0)
            lane = jax.lax.broadcasted_iota(jnp.int32, (_B, _TV), 1)
            cnt0_ref[...] = row * V + lane

            def loop0(b, _):
                xs_ref[b, 0] = xinit_ref[b]
                return _
            jax.lax.fori_loop(0, _B, loop0, None)

        @pl.when(i > 0)
        def _update():
            def loop(b, _):
                xp = xs_ref[b, 0]
                x1 = x1s_ref[b, 0]
                jump = jnp.logical_and(ub_ref[i - 1, b] < thrb_ref[i - 1],
                                       x1 != xp)
                xs_ref[b, 0] = jnp.where(jump, x1, xp)
                return _
            jax.lax.fori_loop(0, _B, loop, None)

        def gather(b, _):
            pltpu.make_async_copy(
                emb_ref.at[pl.ds(xs_ref[b, 0], 1), :],
                h_ref.at[pl.ds(b, 1), :], sem).start()
            return _
        jax.lax.fori_loop(0, _B, gather, None)

        def drain(b, _):
            pltpu.make_async_copy(
                emb_ref.at[pl.ds(0, 1), :],
                h_ref.at[pl.ds(b, 1), :], sem).wait()
            return _
        jax.lax.fori_loop(0, _B, drain, None)
        h_ref[...] = h_ref[...] + tv

        mx_ref[...] = jnp.full((_B, 1), neg_inf, jnp.float32)
        se_ref[...] = jnp.zeros((_B, 1), jnp.float32)
        s1_ref[...] = jnp.full((_B, 1), neg_inf, jnp.float32)
        s2_ref[...] = jnp.full((_B, 1), neg_inf, jnp.float32)
        l1_ref[...] = jnp.zeros((_B, 1), jnp.float32)
        l2_ref[...] = jnp.zeros((_B, 1), jnp.float32)
        i1_ref[...] = jnp.zeros((_B, 1), jnp.int32)
        i2_ref[...] = jnp.zeros((_B, 1), jnp.int32)

    @pl.when(j < T)
    def _sweep():
        cnt0 = cnt0_ref[...]
        g_ref[...] = _gumbel_at(keys_ref[i, 0], keys_ref[i, 1],
                                cnt0.astype(jnp.uint32),
                                j.astype(jnp.uint32) * np.uint32(_TV))

        logits = jnp.dot(h_ref[...], w_ref[...],
                         preferred_element_type=jnp.float32) + b_ref[...]

        tmax = jnp.max(logits, axis=1, keepdims=True)
        old_mx = mx_ref[...]
        new_mx = jnp.maximum(old_mx, tmax)
        e = jnp.exp(logits - new_mx)
        se_ref[...] = se_ref[...] * jnp.exp(old_mx - new_mx) \
            + jnp.sum(e, axis=1, keepdims=True)
        mx_ref[...] = new_mx

        s = g_ref[...] + logits
        rowv = jax.lax.broadcasted_iota(jnp.int32, (_B, 1), 0) * V
        m1 = jnp.max(s, axis=1, keepdims=True)
        a1 = jnp.argmax(s, axis=1).astype(jnp.int32).reshape(_B, 1)
        oh1 = cnt0 == rowv + a1
        lt1 = _pick_at(oh1, logits, neg_inf)
        s_m = jnp.where(oh1, neg_inf, s)
        m2 = jnp.max(s_m, axis=1, keepdims=True)
        a2 = jnp.argmax(s_m, axis=1).astype(jnp.int32).reshape(_B, 1)
        lt2 = _pick_at(cnt0 == rowv + a2, logits, neg_inf)
        ga1 = a1 + j * _TV
        ga2 = a2 + j * _TV

        rs1, rs2 = s1_ref[...], s2_ref[...]
        take1 = m1 > rs1
        n1_s = jnp.where(take1, m1, rs1)
        n1_l = jnp.where(take1, lt1, l1_ref[...])
        n1_i = jnp.where(take1, ga1, i1_ref[...])
        c_s = jnp.where(take1, rs1, rs2)
        c_l = jnp.where(take1, l1_ref[...], l2_ref[...])
        c_i = jnp.where(take1, i1_ref[...], i2_ref[...])
        d_s = jnp.where(take1, m2, m1)
        d_l = jnp.where(take1, lt2, lt1)
        d_i = jnp.where(take1, ga2, ga1)
        take2 = d_s > c_s
        s2_ref[...] = jnp.where(take2, d_s, c_s)
        l2_ref[...] = jnp.where(take2, d_l, c_l)
        i2_ref[...] = jnp.where(take2, d_i, c_i)
        s1_ref[...] = n1_s
        l1_ref[...] = n1_l
        i1_ref[...] = n1_i

    @pl.when(j == T)
    def _finalize():
        mx = mx_ref[...]
        se = se_ref[...]
        eps = jnp.float32(1e-30)
        rowv = jax.lax.broadcasted_iota(jnp.int32, (_B, 1), 0)
        i1 = i1_ref[...]
        i2 = i2_ref[...]
        k0 = keys_ref[i, 0]
        k1 = keys_ref[i, 1]
        g1 = _gumbel_at(k0, k1, (rowv * V + i1).astype(jnp.uint32))
        g2 = _gumbel_at(k0, k1, (rowv * V + i2).astype(jnp.uint32))
        v1 = jnp.log(jnp.exp(l1_ref[...] - mx) / se + eps) + g1
        v2 = jnp.log(jnp.exp(l2_ref[...] - mx) / se + eps) + g2
        pick2 = (v2 > v1) | ((v2 == v1) & (i2 < i1))
        x1 = jnp.where(pick2, i2, i1)

        @pl.when(i < _N - 1)
        def _handoff():
            x1v_ref[...] = x1
            dma = pltpu.make_async_copy(x1v_ref, x1s_ref, sem)
            dma.start()
            dma.wait()

        @pl.when(i == _N - 1)
        def _emit():
            out_ref[...] = x1


def kernel(x_init, emb, W, b, t_w, n_steps):
    B, S = x_init.shape
    V, D = emb.shape
    step_size = 1.0 / _N
    t_disc = jnp.array([step_size * i for i in range(_N)] + [1.0],
                       dtype=jnp.float32)
    t_disc = t_disc * (n_steps / _N)

    T = -(-V // _TV)
    VP = T * _TV
    w_pad = jnp.pad(W, ((0, 0), (0, VP - V)))
    b_pad = jnp.pad(b, (0, VP - V),
                    constant_values=-jnp.inf).reshape(1, VP)

    key = jax.random.key(42)
    keys = []
    unifs = []
    thrs = []
    for i in range(_N):
        key, k_cat, k_jump, k_cat2 = jax.random.split(key, 4)
        keys.append(jax.random.key_data(k_cat))
        if i < _N - 1:
            t = t_disc[i]
            h = t_disc[i + 1] - t_disc[i]
            unifs.append(jax.random.uniform(k_jump, (B, S)).reshape(B))
            intensity = jnp.float32(1.0) / (1.0 - t)
            thrs.append(1.0 - jnp.exp(-h * intensity))
    keys = jnp.stack(keys)
    ub = jax.lax.bitcast_convert_type(
        jnp.stack(unifs + [jnp.zeros(B, jnp.float32)]), jnp.int32)
    thrb = jax.lax.bitcast_convert_type(
        jnp.stack(thrs + [jnp.float32(0.0)]), jnp.int32)
    tvec = (t_disc[:_N, None] * t_w[None, :]).reshape(_N, 1, D)

    f32 = jnp.float32
    out = pl.pallas_call(
        functools.partial(_body, V=V, T=T),
        grid=(_N, T + 1),
        in_specs=[
            pl.BlockSpec(memory_space=pltpu.SMEM),
            pl.BlockSpec(memory_space=pltpu.SMEM),
            pl.BlockSpec(memory_space=pltpu.SMEM),
            pl.BlockSpec(memory_space=pltpu.SMEM),
            pl.BlockSpec(memory_space=pltpu.MemorySpace.HBM),
            pl.BlockSpec((D, _TV),
                         lambda i, j: (0, jnp.minimum(j, T - 1))),
            pl.BlockSpec((1, _TV),
                         lambda i, j: (0, jnp.minimum(j, T - 1))),
            pl.BlockSpec((1, 1, D), lambda i, j: (i, 0, 0)),
        ],
        out_specs=pl.BlockSpec((_B, 1), lambda i, j: (0, 0)),
        out_shape=jax.ShapeDtypeStruct((_B, 1), jnp.int32),
        scratch_shapes=[
            pltpu.VMEM((_B, 1), f32), pltpu.VMEM((_B, 1), f32),
            pltpu.VMEM((_B, 1), f32), pltpu.VMEM((_B, 1), f32),
            pltpu.VMEM((_B, 1), jnp.int32),
            pltpu.VMEM((_B, 1), f32), pltpu.VMEM((_B, 1), f32),
            pltpu.VMEM((_B, 1), jnp.int32),
            pltpu.VMEM((_B, D), f32),
            pltpu.SMEM((_B, 1), jnp.int32),
            pltpu.VMEM((_B, 1), jnp.int32),
            pltpu.SMEM((_B, 1), jnp.int32),
            pltpu.VMEM((_B, _TV), jnp.int32),
            pltpu.VMEM((_B, _TV), jnp.float32),
            pltpu.SemaphoreType.DMA,
        ],
        compiler_params=pltpu.CompilerParams(
            dimension_semantics=("arbitrary", "arbitrary"),
            vmem_limit_bytes=60 * 1024 * 1024),
    )(x_init[:, 0], ub, thrb, keys, emb, w_pad, b_pad, tvec)
    return out
